# baseline (speedup 1.0000x reference)
import jax
import jax.numpy as jnp
from jax import lax
from jax.experimental import pallas as pl
from jax.experimental.pallas import tpu as pltpu
from jax.experimental.pallas import tpu_sc as plsc

_NC = 2
_NS = 16
_NW = _NC * _NS
_W = 160
_NCHUNKS = 40
_PER_W = _W * _NCHUNKS


def kernel(x, table):
    batch, hist = x.shape
    vocab, dim = table.shape
    n = batch * hist
    idx = x.reshape(n).astype(jnp.int32)
    table128 = jnp.pad(table, ((0, 0), (0, 128 - dim)))

    mesh = plsc.VectorSubcoreMesh(core_axis_name="c", subcore_axis_name="s")

    @pl.kernel(
        out_type=jax.ShapeDtypeStruct((n // 2, 2 * dim), table.dtype),
        mesh=mesh,
        scratch_types=[
            pltpu.VMEM((_PER_W,), jnp.int32),
            pltpu.VMEM((_W, 128), table.dtype),
            pltpu.VMEM((_W, 128), table.dtype),
            pltpu.VMEM((_W // 2, 128), table.dtype),
            pltpu.VMEM((_W // 2, 128), table.dtype),
            pltpu.SemaphoreType.DMA,
            pltpu.SemaphoreType.DMA,
        ],
    )
    def gather_kernel(table_hbm, idx_hbm, out_hbm, idx_v, rows0, rows1,
                      pack0, pack1, sem0, sem1):
        wid = lax.axis_index("s") * _NC + lax.axis_index("c")
        base = wid * _PER_W

        pltpu.sync_copy(idx_hbm.at[pl.ds(base, _PER_W)], idx_v)

        def start(g, rows, sem):
            pltpu.async_copy(
                table_hbm.at[idx_v.at[pl.ds(g * _W, _W)]], rows, sem
            )

        def wait(rows, sem):
            pltpu.make_async_copy(table_hbm.at[pl.ds(0, _W)], rows, sem).wait()

        def emit(g, rows, pack):
            @pl.loop(0, _W // 2)
            def _(j):
                for c in range(dim // 16):
                    lo = pl.ds(c * 16, 16)
                    hi = pl.ds(dim + c * 16, 16)
                    pack.at[pl.ds(j, 1), lo][...] = (
                        rows.at[pl.ds(2 * j, 1), lo][...]
                    )
                    pack.at[pl.ds(j, 1), hi][...] = (
                        rows.at[pl.ds(2 * j + 1, 1), lo][...]
                    )

            off = pl.multiple_of((base + g * _W) // 2, 8)
            pltpu.sync_copy(pack, out_hbm.at[pl.ds(off, _W // 2)])

        start(0, rows0, sem0)

        @pl.loop(0, _NCHUNKS - 2, step=2)
        def _(g):
            start(g + 1, rows1, sem1)
            wait(rows0, sem0)
            emit(g, rows0, pack0)
            start(g + 2, rows0, sem0)
            wait(rows1, sem1)
            emit(g + 1, rows1, pack1)

        start(_NCHUNKS - 1, rows1, sem1)
        wait(rows0, sem0)
        emit(_NCHUNKS - 2, rows0, pack0)
        wait(rows1, sem1)
        emit(_NCHUNKS - 1, rows1, pack1)

    out128 = gather_kernel(table128, idx)
    return out128.reshape(batch, hist, dim)
